# baseline (speedup 1.0000x reference)
import functools

import jax
import jax.numpy as jnp
from jax.experimental import pallas as pl
from jax.experimental.pallas import tpu as pltpu
from jax.experimental.pallas import tpu_sc as plsc

T = 2048
D = 1024
F = 512
E = 8
N_GROUP = 4
TOP_K = 2
TOPK_GROUP = 2
ROUTED_SCALING_FACTOR = 2.5
EPAD = 128
BT = 256
NB = T * TOP_K // BT + E
NP = NB * BT
SCW = 128
DH = D // 2
DH2 = DH // 2


def _gate_kernel(lg_ref, bias_ref, meta_ref, blk_ref):
    logits = lg_ref[...]
    scores = jax.nn.sigmoid(logits)
    biased = scores + bias_ref[...]

    lane_r = jax.lax.broadcasted_iota(jnp.int32, (EPAD, EPAD), 0)
    lane_c = jax.lax.broadcasted_iota(jnp.int32, (EPAD, EPAD), 1)
    lane1 = jax.lax.broadcasted_iota(jnp.int32, (1, EPAD), 1)

    f32 = jnp.float32
    hi = jax.lax.Precision.HIGHEST

    def pairwise_topk_mask(vals, n, k):
        A = ((lane_c // n) == lane_r).astype(f32)
        B = ((lane_c % n) == lane_r).astype(f32)
        X = jnp.dot(vals, A, preferred_element_type=f32, precision=hi)
        Y = jnp.dot(vals, B, preferred_element_type=f32, precision=hi)
        tie = ((lane1 % n) < (lane1 // n)).astype(f32)
        valid = (lane1 < n * n) & ((lane1 % n) != (lane1 // n))
        beats = jnp.where((Y > X) | ((Y == X) & (tie > 0)), 1.0, 0.0)
        beats = jnp.where(valid, beats, 0.0)
        Csum = ((lane_r // n) == lane_c).astype(f32) * \
               (lane_r < n * n).astype(f32)
        rank = jnp.dot(beats, Csum, preferred_element_type=f32, precision=hi)
        return jnp.where((rank < k) & (lane1 < n), 1.0, 0.0)

    P = (((lane_r // 2) == lane_c) & (lane_r < E)).astype(f32)
    gscore = jnp.dot(biased, P, preferred_element_type=f32, precision=hi)
    gsel = pairwise_topk_mask(gscore, N_GROUP, TOPK_GROUP)
    Q = ((lane_r == (lane_c // 2)) & (lane_c < E)).astype(f32)
    em = jnp.dot(gsel, Q, preferred_element_type=f32, precision=hi)
    masked = jnp.where((em > 0) & (lane1 < E), biased, -1e9)
    sel = pairwise_topk_mask(masked, E, TOP_K)

    picked = sel * scores
    Ones8 = ((lane_r < E) & (lane_c < E)).astype(f32)
    wsum = jnp.dot(picked, Ones8, preferred_element_type=f32, precision=hi)
    rw = picked * (ROUTED_SCALING_FACTOR / (wsum + 1e-20))

    rT = jax.lax.broadcasted_iota(jnp.int32, (T, T), 0)
    cT = jax.lax.broadcasted_iota(jnp.int32, (T, T), 1)
    Lst = (cT < rT).astype(jnp.bfloat16)
    csum = jnp.dot(Lst, sel.astype(jnp.bfloat16),
                   preferred_element_type=f32)
    counts = jnp.sum(sel, axis=0, keepdims=True)
    pc = jnp.ceil(counts * (1.0 / BT)) * BT
    LTI = ((lane_r <= lane_c) & (lane_r < E)).astype(f32)
    ends = jnp.dot(pc, LTI, preferred_element_type=f32, precision=hi)
    offs = ends - pc
    pos = offs + csum

    lane_f = lane1.astype(f32)
    e0 = jnp.min(jnp.where(sel > 0, lane_f, 1e9), axis=1, keepdims=True)
    e1 = jnp.max(jnp.where(sel > 0, lane_f, -1.0), axis=1, keepdims=True)
    m0 = (lane_f == e0).astype(f32)
    m1 = (lane_f == e1).astype(f32)
    p0 = jnp.sum(pos * m0, axis=1, keepdims=True)
    p1 = jnp.sum(pos * m1, axis=1, keepdims=True)
    w0 = jnp.sum(rw * m0, axis=1, keepdims=True)
    w1 = jnp.sum(rw * m1, axis=1, keepdims=True)

    meta_ref[...] = (p0 * (lane1 == 0) + p1 * (lane1 == 1) +
                     w0 * (lane1 == 2) + w1 * (lane1 == 3))

    bstart = lane_f * BT
    emap = jnp.zeros((1, EPAD), f32)
    for e in range(E):
        emap += (bstart >= ends[:, e:e + 1]).astype(f32)
    total = ends[:, E - 1:E]
    brow = jax.lax.broadcasted_iota(jnp.int32, (8, EPAD), 0)
    blk_ref[...] = jnp.where(brow == 0, jnp.minimum(emap, E - 1.0),
                             jnp.where(brow == 1,
                                       (bstart < total).astype(f32), 0.0))


def _routed_kernel(emap_ref, valid_ref, xga_ref, xgb_ref,
                   w1_ref, w3_ref, w2_ref,
                   yga_ref, ygb_ref, w1b, w3b, w2b):
    b = pl.program_id(0)
    changed = (b == 0) | (emap_ref[b] != emap_ref[jnp.maximum(b - 1, 0)])

    @pl.when(changed)
    def _cast():
        w1b[...] = w1_ref[0].astype(jnp.bfloat16)
        w3b[...] = w3_ref[0].astype(jnp.bfloat16)
        w2b[...] = w2_ref[0].astype(jnp.bfloat16)

    @pl.when(valid_ref[b] == 1)
    def _compute():
        xa = xga_ref[...]
        xb = xgb_ref[...]
        f32 = jnp.float32
        h1 = (jnp.dot(xa, w1b[:DH], preferred_element_type=f32) +
              jnp.dot(xb, w1b[DH:], preferred_element_type=f32))
        h3 = (jnp.dot(xa, w3b[:DH], preferred_element_type=f32) +
              jnp.dot(xb, w3b[DH:], preferred_element_type=f32))
        h = (jax.nn.silu(h1) * h3).astype(jnp.bfloat16)
        y = jnp.dot(h, w2b[...], preferred_element_type=f32)
        yga_ref[...] = y[:, :DH].astype(jnp.bfloat16)
        ygb_ref[...] = y[:, DH:].astype(jnp.bfloat16)


def _shared_kernel(x_ref, w1_ref, w3_ref, w2_ref, out_ref, w1b, w3b, w2b):
    e = pl.program_id(0)

    @pl.when(e == 0)
    def _init():
        out_ref[...] = jnp.zeros_like(out_ref)

    w1b[...] = w1_ref[0].astype(jnp.bfloat16)
    w3b[...] = w3_ref[0].astype(jnp.bfloat16)
    w2b[...] = w2_ref[0].astype(jnp.bfloat16)
    xs = x_ref[...]
    h1 = jnp.dot(xs, w1b[...], preferred_element_type=jnp.float32)
    h3 = jnp.dot(xs, w3b[...], preferred_element_type=jnp.float32)
    h = jax.nn.silu(h1) * h3
    out_ref[...] += jnp.dot(h.astype(jnp.bfloat16), w2b[...],
                            preferred_element_type=jnp.float32)


def _combine_kernel(ya_ref, yb_ref, sh_ref, meta_ref, out_ref):
    w0 = meta_ref[:, 2:3]
    w1 = meta_ref[:, 3:4]
    f32 = jnp.float32
    left = (sh_ref[:, :DH] + w0 * ya_ref[0].astype(f32) +
            w1 * ya_ref[1].astype(f32))
    right = (sh_ref[:, DH:] + w0 * yb_ref[0].astype(f32) +
             w1 * yb_ref[1].astype(f32))
    out_ref[:, :DH] = left
    out_ref[:, DH:] = right


def _b2i(a):
    return jax.lax.bitcast_convert_type(
        a.reshape(a.shape[0], -1, 2), jnp.int32)


def _i2b(a):
    b = jax.lax.bitcast_convert_type(a, jnp.bfloat16)
    return b.reshape(*a.shape[:-1], a.shape[-1] * 2)


def _sc_scatter(xa, xb, p01):
    vmesh = plsc.VectorSubcoreMesh(core_axis_name="c", subcore_axis_name="s")

    @pl.kernel(out_type=(jax.ShapeDtypeStruct((NP, DH2), jnp.int32),
                         jax.ShapeDtypeStruct((NP, DH2), jnp.int32)),
               mesh=vmesh)
    def kern(xa_hbm, xb_hbm, i_hbm, xga_hbm, xgb_hbm):
        def body_a(x_vmem, i_vmem):
            pltpu.sync_copy(x_vmem, xga_hbm.at[i_vmem.at[0]])

        def body_b(x_vmem, i_vmem):
            pltpu.sync_copy(x_vmem, xgb_hbm.at[i_vmem.at[0]])

        for body, src in ((body_a, xa_hbm), (body_b, xb_hbm)):
            pltpu.emit_pipeline(
                body,
                grid=(TOP_K, T // SCW),
                in_specs=[pl.BlockSpec((SCW, DH2), lambda k, i: (i, 0)),
                          pl.BlockSpec((1, SCW), lambda k, i: (k, i))],
                out_specs=[],
                core_axis_name=("c", "s"),
                dimension_semantics=(pltpu.PARALLEL, pltpu.PARALLEL),
            )(src, i_hbm)

    return kern(xa, xb, p01)


def _sc_gather(yga, ygb, p01):
    vmesh = plsc.VectorSubcoreMesh(core_axis_name="c", subcore_axis_name="s")

    @pl.kernel(out_type=(jax.ShapeDtypeStruct((TOP_K, T, DH2), jnp.int32),
                         jax.ShapeDtypeStruct((TOP_K, T, DH2), jnp.int32)),
               mesh=vmesh)
    def kern(yga_hbm, ygb_hbm, i_hbm, ya_hbm, yb_hbm):
        def body_a(i_vmem, o_vmem):
            pltpu.sync_copy(yga_hbm.at[i_vmem.at[0]], o_vmem.at[0])

        def body_b(i_vmem, o_vmem):
            pltpu.sync_copy(ygb_hbm.at[i_vmem.at[0]], o_vmem.at[0])

        for body, dst in ((body_a, ya_hbm), (body_b, yb_hbm)):
            pltpu.emit_pipeline(
                body,
                grid=(TOP_K, T // SCW),
                in_specs=[pl.BlockSpec((1, SCW), lambda k, i: (k, i))],
                out_specs=[pl.BlockSpec((1, SCW, DH2),
                                        lambda k, i: (k, i, 0))],
                core_axis_name=("c", "s"),
                dimension_semantics=(pltpu.PARALLEL, pltpu.PARALLEL),
            )(i_hbm, dst)

    return kern(yga, ygb, p01)


@functools.partial(jax.jit, static_argnames=("interpret",))
def _moe(x, gate_weight, bias, W1, W3, W2, Ws1, Ws3, Ws2, interpret=False):
    logits = jnp.matmul(x, gate_weight.T)
    lg = jnp.zeros((T, EPAD), jnp.float32).at[:, :E].set(logits)
    bias_row = jnp.zeros((1, EPAD), jnp.float32).at[0, :E].set(bias)

    meta, blk = pl.pallas_call(
        _gate_kernel,
        out_shape=(jax.ShapeDtypeStruct((T, EPAD), jnp.float32),
                   jax.ShapeDtypeStruct((8, EPAD), jnp.float32)),
        interpret=interpret,
    )(lg, bias_row)

    p01 = meta[:, :TOP_K].T.astype(jnp.int32)
    emap_s = blk[0, :NB].astype(jnp.int32)
    valid_s = blk[1, :NB].astype(jnp.int32)

    x_bf = x.astype(jnp.bfloat16)

    FS = Ws1.shape[1]
    ns = FS // F
    Ws1r = Ws1.reshape(D, ns, F).transpose(1, 0, 2)
    Ws3r = Ws3.reshape(D, ns, F).transpose(1, 0, 2)
    Ws2r = Ws2.reshape(ns, F, D)
    shared = pl.pallas_call(
        _shared_kernel,
        grid=(ns,),
        in_specs=[
            pl.BlockSpec((T, D), lambda e: (0, 0)),
            pl.BlockSpec((1, D, F), lambda e: (e, 0, 0)),
            pl.BlockSpec((1, D, F), lambda e: (e, 0, 0)),
            pl.BlockSpec((1, F, D), lambda e: (e, 0, 0)),
        ],
        out_specs=pl.BlockSpec((T, D), lambda e: (0, 0)),
        out_shape=jax.ShapeDtypeStruct((T, D), jnp.float32),
        scratch_shapes=[
            pltpu.VMEM((D, F), jnp.bfloat16),
            pltpu.VMEM((D, F), jnp.bfloat16),
            pltpu.VMEM((F, D), jnp.bfloat16),
        ],
        interpret=interpret,
    )(x_bf, Ws1r, Ws3r, Ws2r)

    xga32, xgb32 = _sc_scatter(_b2i(x_bf[:, :DH]), _b2i(x_bf[:, DH:]), p01)
    xga, xgb = _i2b(xga32), _i2b(xgb32)

    yga, ygb = pl.pallas_call(
        _routed_kernel,
        grid_spec=pltpu.PrefetchScalarGridSpec(
            num_scalar_prefetch=2,
            grid=(NB,),
            in_specs=[
                pl.BlockSpec((BT, DH), lambda b, em, va: (b, 0)),
                pl.BlockSpec((BT, DH), lambda b, em, va: (b, 0)),
                pl.BlockSpec((1, D, F), lambda b, em, va: (em[b], 0, 0)),
                pl.BlockSpec((1, D, F), lambda b, em, va: (em[b], 0, 0)),
                pl.BlockSpec((1, F, D), lambda b, em, va: (em[b], 0, 0)),
            ],
            out_specs=(pl.BlockSpec((BT, DH), lambda b, em, va: (b, 0)),
                       pl.BlockSpec((BT, DH), lambda b, em, va: (b, 0))),
            scratch_shapes=[
                pltpu.VMEM((D, F), jnp.bfloat16),
                pltpu.VMEM((D, F), jnp.bfloat16),
                pltpu.VMEM((F, D), jnp.bfloat16),
            ],
        ),
        out_shape=(jax.ShapeDtypeStruct((NP, DH), jnp.bfloat16),
                   jax.ShapeDtypeStruct((NP, DH), jnp.bfloat16)),
        interpret=interpret,
    )(emap_s, valid_s, xga, xgb, W1, W3, W2)

    ya32, yb32 = _sc_gather(_b2i(yga), _b2i(ygb), p01)
    ya, yb = _i2b(ya32), _i2b(yb32)

    NTB = 4
    out = pl.pallas_call(
        _combine_kernel,
        grid=(NTB,),
        in_specs=[
            pl.BlockSpec((TOP_K, T // NTB, DH), lambda t: (0, t, 0)),
            pl.BlockSpec((TOP_K, T // NTB, DH), lambda t: (0, t, 0)),
            pl.BlockSpec((T // NTB, D), lambda t: (t, 0)),
            pl.BlockSpec((T // NTB, EPAD), lambda t: (t, 0)),
        ],
        out_specs=pl.BlockSpec((T // NTB, D), lambda t: (t, 0)),
        out_shape=jax.ShapeDtypeStruct((T, D), jnp.float32),
        interpret=interpret,
    )(ya, yb, shared, meta)
    return out


def kernel(hidden_states, gate_weight, e_score_correction_bias,
           W1, W3, W2, Ws1, Ws3, Ws2):
    return _moe(hidden_states, gate_weight, e_score_correction_bias,
                W1, W3, W2, Ws1, Ws3, Ws2)


# speedup vs baseline: 3.1507x; 3.1507x over previous
import functools

import jax
import jax.numpy as jnp
from jax.experimental import pallas as pl
from jax.experimental.pallas import tpu as pltpu
from jax.experimental.pallas import tpu_sc as plsc

T = 2048
D = 1024
F = 512
E = 8
N_GROUP = 4
TOP_K = 2
TOPK_GROUP = 2
ROUTED_SCALING_FACTOR = 2.5
EPAD = 128
BT = 256
NB = T * TOP_K // BT + E
NP = NB * BT
SCW = 128
NQ = 4
DQ = D // NQ


def _gate_kernel(lg_ref, bias_ref, meta_ref, blk_ref):
    logits = lg_ref[...]
    scores = jax.nn.sigmoid(logits)
    biased = scores + bias_ref[...]

    lane_r = jax.lax.broadcasted_iota(jnp.int32, (EPAD, EPAD), 0)
    lane_c = jax.lax.broadcasted_iota(jnp.int32, (EPAD, EPAD), 1)
    lane1 = jax.lax.broadcasted_iota(jnp.int32, (1, EPAD), 1)

    f32 = jnp.float32
    hi = jax.lax.Precision.HIGHEST

    def pairwise_topk_mask(vals, n, k):
        A = ((lane_c // n) == lane_r).astype(f32)
        B = ((lane_c % n) == lane_r).astype(f32)
        X = jnp.dot(vals, A, preferred_element_type=f32, precision=hi)
        Y = jnp.dot(vals, B, preferred_element_type=f32, precision=hi)
        tie = ((lane1 % n) < (lane1 // n)).astype(f32)
        valid = (lane1 < n * n) & ((lane1 % n) != (lane1 // n))
        beats = jnp.where((Y > X) | ((Y == X) & (tie > 0)), 1.0, 0.0)
        beats = jnp.where(valid, beats, 0.0)
        Csum = ((lane_r // n) == lane_c).astype(f32) * \
               (lane_r < n * n).astype(f32)
        rank = jnp.dot(beats, Csum, preferred_element_type=f32, precision=hi)
        return jnp.where((rank < k) & (lane1 < n), 1.0, 0.0)

    P = (((lane_r // 2) == lane_c) & (lane_r < E)).astype(f32)
    gscore = jnp.dot(biased, P, preferred_element_type=f32, precision=hi)
    gsel = pairwise_topk_mask(gscore, N_GROUP, TOPK_GROUP)
    Q = ((lane_r == (lane_c // 2)) & (lane_c < E)).astype(f32)
    em = jnp.dot(gsel, Q, preferred_element_type=f32, precision=hi)
    masked = jnp.where((em > 0) & (lane1 < E), biased, -1e9)
    sel = pairwise_topk_mask(masked, E, TOP_K)

    picked = sel * scores
    Ones8 = ((lane_r < E) & (lane_c < E)).astype(f32)
    wsum = jnp.dot(picked, Ones8, preferred_element_type=f32, precision=hi)
    rw = picked * (ROUTED_SCALING_FACTOR / (wsum + 1e-20))

    rT = jax.lax.broadcasted_iota(jnp.int32, (T, T), 0)
    cT = jax.lax.broadcasted_iota(jnp.int32, (T, T), 1)
    Lst = (cT < rT).astype(jnp.bfloat16)
    csum = jnp.dot(Lst, sel.astype(jnp.bfloat16),
                   preferred_element_type=f32)
    counts = jnp.sum(sel, axis=0, keepdims=True)
    pc = jnp.ceil(counts * (1.0 / BT)) * BT
    LTI = ((lane_r <= lane_c) & (lane_r < E)).astype(f32)
    ends = jnp.dot(pc, LTI, preferred_element_type=f32, precision=hi)
    offs = ends - pc
    pos = offs + csum

    lane_f = lane1.astype(f32)
    e0 = jnp.min(jnp.where(sel > 0, lane_f, 1e9), axis=1, keepdims=True)
    e1 = jnp.max(jnp.where(sel > 0, lane_f, -1.0), axis=1, keepdims=True)
    m0 = (lane_f == e0).astype(f32)
    m1 = (lane_f == e1).astype(f32)
    p0 = jnp.sum(pos * m0, axis=1, keepdims=True)
    p1 = jnp.sum(pos * m1, axis=1, keepdims=True)
    w0 = jnp.sum(rw * m0, axis=1, keepdims=True)
    w1 = jnp.sum(rw * m1, axis=1, keepdims=True)

    meta_ref[...] = (p0 * (lane1 == 0) + p1 * (lane1 == 1) +
                     w0 * (lane1 == 2) + w1 * (lane1 == 3))

    bstart = lane_f * BT
    emap = jnp.zeros((1, EPAD), f32)
    for e in range(E):
        emap += (bstart >= ends[:, e:e + 1]).astype(f32)
    total = ends[:, E - 1:E]
    brow = jax.lax.broadcasted_iota(jnp.int32, (8, EPAD), 0)
    blk_ref[...] = jnp.where(brow == 0, jnp.minimum(emap, E - 1.0),
                             jnp.where(brow == 1,
                                       (bstart < total).astype(f32), 0.0))


def _routed_kernel(emap_ref, valid_ref, xg0_ref, xg1_ref, xg2_ref, xg3_ref,
                   w1_ref, w3_ref, w2_ref,
                   yg0_ref, yg1_ref, yg2_ref, yg3_ref, w1b, w3b, w2b):
    b = pl.program_id(0)
    changed = (b == 0) | (emap_ref[b] != emap_ref[jnp.maximum(b - 1, 0)])

    @pl.when(changed)
    def _cast():
        w1b[...] = w1_ref[0].astype(jnp.bfloat16)
        w3b[...] = w3_ref[0].astype(jnp.bfloat16)
        w2b[...] = w2_ref[0].astype(jnp.bfloat16)

    @pl.when(valid_ref[b] == 1)
    def _compute():
        f32 = jnp.float32
        xq = [xg0_ref, xg1_ref, xg2_ref, xg3_ref]
        h1 = jnp.zeros((BT, F), f32)
        h3 = jnp.zeros((BT, F), f32)
        for q in range(NQ):
            xb = xq[q][...].astype(jnp.bfloat16)
            h1 += jnp.dot(xb, w1b[q * DQ:(q + 1) * DQ],
                          preferred_element_type=f32)
            h3 += jnp.dot(xb, w3b[q * DQ:(q + 1) * DQ],
                          preferred_element_type=f32)
        h = (jax.nn.silu(h1) * h3).astype(jnp.bfloat16)
        y = jnp.dot(h, w2b[...], preferred_element_type=f32)
        yq = [yg0_ref, yg1_ref, yg2_ref, yg3_ref]
        for q in range(NQ):
            yq[q][...] = y[:, q * DQ:(q + 1) * DQ]


def _shared_kernel(x_ref, w1_ref, w3_ref, w2_ref, out_ref, w1b, w3b, w2b):
    e = pl.program_id(0)

    @pl.when(e == 0)
    def _init():
        out_ref[...] = jnp.zeros_like(out_ref)

    w1b[...] = w1_ref[0].astype(jnp.bfloat16)
    w3b[...] = w3_ref[0].astype(jnp.bfloat16)
    w2b[...] = w2_ref[0].astype(jnp.bfloat16)
    xs = x_ref[...]
    h1 = jnp.dot(xs, w1b[...], preferred_element_type=jnp.float32)
    h3 = jnp.dot(xs, w3b[...], preferred_element_type=jnp.float32)
    h = jax.nn.silu(h1) * h3
    out_ref[...] += jnp.dot(h.astype(jnp.bfloat16), w2b[...],
                            preferred_element_type=jnp.float32)


def _combine_kernel(y0_ref, y1_ref, y2_ref, y3_ref, sh_ref, meta_ref,
                    out_ref):
    w0 = meta_ref[:, 2:3]
    w1 = meta_ref[:, 3:4]
    yq = [y0_ref, y1_ref, y2_ref, y3_ref]
    for q in range(NQ):
        sl = slice(q * DQ, (q + 1) * DQ)
        out_ref[:, sl] = (sh_ref[:, sl] + w0 * yq[q][0] + w1 * yq[q][1])


def _sc_scatter(x, p01):
    vmesh = plsc.VectorSubcoreMesh(core_axis_name="c", subcore_axis_name="s")

    @pl.kernel(out_type=tuple(jax.ShapeDtypeStruct((NP, DQ), jnp.float32)
                              for _ in range(NQ)),
               mesh=vmesh)
    def kern(x_hbm, i_hbm, *xg_hbm):
        for q in range(NQ):
            dst = xg_hbm[q]

            def body(x_vmem, i_vmem, dst=dst):
                pltpu.sync_copy(x_vmem, dst.at[i_vmem.at[0]])

            pltpu.emit_pipeline(
                body,
                grid=(TOP_K, T // SCW),
                in_specs=[pl.BlockSpec((SCW, DQ), lambda k, i, q=q: (i, q)),
                          pl.BlockSpec((1, SCW), lambda k, i: (k, i))],
                out_specs=[],
                core_axis_name=("c", "s"),
                dimension_semantics=(pltpu.PARALLEL, pltpu.PARALLEL),
            )(x_hbm, i_hbm)

    return kern(x, p01)


def _sc_gather(ygs, p01):
    vmesh = plsc.VectorSubcoreMesh(core_axis_name="c", subcore_axis_name="s")

    @pl.kernel(out_type=tuple(
        jax.ShapeDtypeStruct((TOP_K, T, DQ), jnp.float32)
        for _ in range(NQ)),
        mesh=vmesh)
    def kern(yg0, yg1, yg2, yg3, i_hbm, *y_hbm):
        srcs = (yg0, yg1, yg2, yg3)
        for q in range(NQ):
            src = srcs[q]

            def body(i_vmem, o_vmem, src=src):
                pltpu.sync_copy(src.at[i_vmem.at[0]], o_vmem.at[0])

            pltpu.emit_pipeline(
                body,
                grid=(TOP_K, T // SCW),
                in_specs=[pl.BlockSpec((1, SCW), lambda k, i: (k, i))],
                out_specs=[pl.BlockSpec((1, SCW, DQ),
                                        lambda k, i: (k, i, 0))],
                core_axis_name=("c", "s"),
                dimension_semantics=(pltpu.PARALLEL, pltpu.PARALLEL),
            )(i_hbm, y_hbm[q])

    return kern(*ygs, p01)


@functools.partial(jax.jit, static_argnames=("interpret",))
def _moe(x, gate_weight, bias, W1, W3, W2, Ws1, Ws3, Ws2, interpret=False):
    logits = jnp.matmul(x, gate_weight.T)
    lg = jnp.zeros((T, EPAD), jnp.float32).at[:, :E].set(logits)
    bias_row = jnp.zeros((1, EPAD), jnp.float32).at[0, :E].set(bias)

    meta, blk = pl.pallas_call(
        _gate_kernel,
        out_shape=(jax.ShapeDtypeStruct((T, EPAD), jnp.float32),
                   jax.ShapeDtypeStruct((8, EPAD), jnp.float32)),
        interpret=interpret,
    )(lg, bias_row)

    p01 = meta[:, :TOP_K].T.astype(jnp.int32)
    emap_s = blk[0, :NB].astype(jnp.int32)
    valid_s = blk[1, :NB].astype(jnp.int32)

    x_bf = x.astype(jnp.bfloat16)

    FS = Ws1.shape[1]
    ns = FS // F
    Ws1r = Ws1.reshape(D, ns, F).transpose(1, 0, 2)
    Ws3r = Ws3.reshape(D, ns, F).transpose(1, 0, 2)
    Ws2r = Ws2.reshape(ns, F, D)
    shared = pl.pallas_call(
        _shared_kernel,
        grid=(ns,),
        in_specs=[
            pl.BlockSpec((T, D), lambda e: (0, 0)),
            pl.BlockSpec((1, D, F), lambda e: (e, 0, 0)),
            pl.BlockSpec((1, D, F), lambda e: (e, 0, 0)),
            pl.BlockSpec((1, F, D), lambda e: (e, 0, 0)),
        ],
        out_specs=pl.BlockSpec((T, D), lambda e: (0, 0)),
        out_shape=jax.ShapeDtypeStruct((T, D), jnp.float32),
        scratch_shapes=[
            pltpu.VMEM((D, F), jnp.bfloat16),
            pltpu.VMEM((D, F), jnp.bfloat16),
            pltpu.VMEM((F, D), jnp.bfloat16),
        ],
        interpret=interpret,
    )(x_bf, Ws1r, Ws3r, Ws2r)

    xgs = _sc_scatter(x, p01)

    xq_spec = pl.BlockSpec((BT, DQ), lambda b, em, va: (b, 0))
    ygs = pl.pallas_call(
        _routed_kernel,
        grid_spec=pltpu.PrefetchScalarGridSpec(
            num_scalar_prefetch=2,
            grid=(NB,),
            in_specs=[
                xq_spec, xq_spec, xq_spec, xq_spec,
                pl.BlockSpec((1, D, F), lambda b, em, va: (em[b], 0, 0)),
                pl.BlockSpec((1, D, F), lambda b, em, va: (em[b], 0, 0)),
                pl.BlockSpec((1, F, D), lambda b, em, va: (em[b], 0, 0)),
            ],
            out_specs=tuple(pl.BlockSpec((BT, DQ), lambda b, em, va: (b, 0))
                            for _ in range(NQ)),
            scratch_shapes=[
                pltpu.VMEM((D, F), jnp.bfloat16),
                pltpu.VMEM((D, F), jnp.bfloat16),
                pltpu.VMEM((F, D), jnp.bfloat16),
            ],
        ),
        out_shape=tuple(jax.ShapeDtypeStruct((NP, DQ), jnp.float32)
                        for _ in range(NQ)),
        interpret=interpret,
    )(emap_s, valid_s, *xgs, W1, W3, W2)

    ys = _sc_gather(ygs, p01)

    NTB = 4
    yq_spec = pl.BlockSpec((TOP_K, T // NTB, DQ), lambda t: (0, t, 0))
    out = pl.pallas_call(
        _combine_kernel,
        grid=(NTB,),
        in_specs=[
            yq_spec, yq_spec, yq_spec, yq_spec,
            pl.BlockSpec((T // NTB, D), lambda t: (t, 0)),
            pl.BlockSpec((T // NTB, EPAD), lambda t: (t, 0)),
        ],
        out_specs=pl.BlockSpec((T // NTB, D), lambda t: (t, 0)),
        out_shape=jax.ShapeDtypeStruct((T, D), jnp.float32),
        interpret=interpret,
    )(*ys, shared, meta)
    return out


def kernel(hidden_states, gate_weight, e_score_correction_bias,
           W1, W3, W2, Ws1, Ws3, Ws2):
    return _moe(hidden_states, gate_weight, e_score_correction_bias,
                W1, W3, W2, Ws1, Ws3, Ws2)


# speedup vs baseline: 3.5686x; 1.1327x over previous
import functools

import jax
import jax.numpy as jnp
from jax.experimental import pallas as pl
from jax.experimental.pallas import tpu as pltpu
from jax.experimental.pallas import tpu_sc as plsc

T = 2048
D = 1024
F = 512
E = 8
N_GROUP = 4
TOP_K = 2
TOPK_GROUP = 2
ROUTED_SCALING_FACTOR = 2.5
EPAD = 128
BT = 256
NB = T * TOP_K // BT + E
NP = NB * BT
SCW = 128
DP = D // 2
DPH = DP // 2


def _pack(lo_bf, hi_bf):
    lo = jax.lax.bitcast_convert_type(lo_bf, jnp.uint16).astype(jnp.uint32)
    hi = jax.lax.bitcast_convert_type(hi_bf, jnp.uint16).astype(jnp.uint32)
    return jax.lax.bitcast_convert_type(lo | (hi << 16), jnp.int32)


def _unpack(w):
    u = jax.lax.bitcast_convert_type(w, jnp.uint32)
    lo = jax.lax.bitcast_convert_type((u & 0xFFFF).astype(jnp.uint16),
                                      jnp.bfloat16)
    hi = jax.lax.bitcast_convert_type((u >> 16).astype(jnp.uint16),
                                      jnp.bfloat16)
    return lo, hi


def _gate_kernel(lg_ref, bias_ref, x_ref, meta_ref, blk_ref, xp_ref):
    logits = lg_ref[...]
    scores = jax.nn.sigmoid(logits)
    biased = scores + bias_ref[...]

    lane_r = jax.lax.broadcasted_iota(jnp.int32, (EPAD, EPAD), 0)
    lane_c = jax.lax.broadcasted_iota(jnp.int32, (EPAD, EPAD), 1)
    lane1 = jax.lax.broadcasted_iota(jnp.int32, (1, EPAD), 1)

    f32 = jnp.float32
    hi = jax.lax.Precision.HIGHEST

    def pairwise_topk_mask(vals, n, k):
        A = ((lane_c // n) == lane_r).astype(f32)
        B = ((lane_c % n) == lane_r).astype(f32)
        X = jnp.dot(vals, A, preferred_element_type=f32, precision=hi)
        Y = jnp.dot(vals, B, preferred_element_type=f32, precision=hi)
        tie = ((lane1 % n) < (lane1 // n)).astype(f32)
        valid = (lane1 < n * n) & ((lane1 % n) != (lane1 // n))
        beats = jnp.where((Y > X) | ((Y == X) & (tie > 0)), 1.0, 0.0)
        beats = jnp.where(valid, beats, 0.0)
        Csum = ((lane_r // n) == lane_c).astype(f32) * \
               (lane_r < n * n).astype(f32)
        rank = jnp.dot(beats, Csum, preferred_element_type=f32, precision=hi)
        return jnp.where((rank < k) & (lane1 < n), 1.0, 0.0)

    P = (((lane_r // 2) == lane_c) & (lane_r < E)).astype(f32)
    gscore = jnp.dot(biased, P, preferred_element_type=f32, precision=hi)
    gsel = pairwise_topk_mask(gscore, N_GROUP, TOPK_GROUP)
    Q = ((lane_r == (lane_c // 2)) & (lane_c < E)).astype(f32)
    em = jnp.dot(gsel, Q, preferred_element_type=f32, precision=hi)
    masked = jnp.where((em > 0) & (lane1 < E), biased, -1e9)
    sel = pairwise_topk_mask(masked, E, TOP_K)

    picked = sel * scores
    Ones8 = ((lane_r < E) & (lane_c < E)).astype(f32)
    wsum = jnp.dot(picked, Ones8, preferred_element_type=f32, precision=hi)
    rw = picked * (ROUTED_SCALING_FACTOR / (wsum + 1e-20))

    rT = jax.lax.broadcasted_iota(jnp.int32, (T, T), 0)
    cT = jax.lax.broadcasted_iota(jnp.int32, (T, T), 1)
    Lst = (cT < rT).astype(jnp.bfloat16)
    csum = jnp.dot(Lst, sel.astype(jnp.bfloat16),
                   preferred_element_type=f32)
    counts = jnp.sum(sel, axis=0, keepdims=True)
    pc = jnp.ceil(counts * (1.0 / BT)) * BT
    LTI = ((lane_r <= lane_c) & (lane_r < E)).astype(f32)
    ends = jnp.dot(pc, LTI, preferred_element_type=f32, precision=hi)
    offs = ends - pc
    pos = offs + csum

    lane_f = lane1.astype(f32)
    e0 = jnp.min(jnp.where(sel > 0, lane_f, 1e9), axis=1, keepdims=True)
    e1 = jnp.max(jnp.where(sel > 0, lane_f, -1.0), axis=1, keepdims=True)
    m0 = (lane_f == e0).astype(f32)
    m1 = (lane_f == e1).astype(f32)
    p0 = jnp.sum(pos * m0, axis=1, keepdims=True)
    p1 = jnp.sum(pos * m1, axis=1, keepdims=True)
    w0 = jnp.sum(rw * m0, axis=1, keepdims=True)
    w1 = jnp.sum(rw * m1, axis=1, keepdims=True)

    meta_ref[...] = (p0 * (lane1 == 0) + p1 * (lane1 == 1) +
                     w0 * (lane1 == 2) + w1 * (lane1 == 3))

    bstart = lane_f * BT
    emap = jnp.zeros((1, EPAD), f32)
    for e in range(E):
        emap += (bstart >= ends[:, e:e + 1]).astype(f32)
    total = ends[:, E - 1:E]
    brow = jax.lax.broadcasted_iota(jnp.int32, (8, EPAD), 0)
    blk_ref[...] = jnp.where(brow == 0, jnp.minimum(emap, E - 1.0),
                             jnp.where(brow == 1,
                                       (bstart < total).astype(f32), 0.0))

    xv = x_ref[...]
    xp_ref[...] = _pack(xv[:, :DP].astype(jnp.bfloat16),
                        xv[:, DP:].astype(jnp.bfloat16))


def _routed_kernel(emap_ref, valid_ref, xg0_ref, xg1_ref,
                   w1_ref, w3_ref, w2_ref,
                   yg0_ref, yg1_ref, w1b, w3b, w2b):
    b = pl.program_id(0)
    changed = (b == 0) | (emap_ref[b] != emap_ref[jnp.maximum(b - 1, 0)])

    @pl.when(changed)
    def _cast():
        w1b[...] = w1_ref[0].astype(jnp.bfloat16)
        w3b[...] = w3_ref[0].astype(jnp.bfloat16)
        w2b[...] = w2_ref[0].astype(jnp.bfloat16)

    @pl.when(valid_ref[b] == 1)
    def _compute():
        f32 = jnp.float32
        lo0, hi0 = _unpack(xg0_ref[...])
        lo1, hi1 = _unpack(xg1_ref[...])
        parts = ((lo0, 0), (lo1, DPH), (hi0, DP), (hi1, DP + DPH))
        h1 = jnp.zeros((BT, F), f32)
        h3 = jnp.zeros((BT, F), f32)
        for xb, c in parts:
            h1 += jnp.dot(xb, w1b[c:c + DPH], preferred_element_type=f32)
            h3 += jnp.dot(xb, w3b[c:c + DPH], preferred_element_type=f32)
        h = (jax.nn.silu(h1) * h3).astype(jnp.bfloat16)
        y = jnp.dot(h, w2b[...], preferred_element_type=f32)
        yp = _pack(y[:, :DP].astype(jnp.bfloat16),
                   y[:, DP:].astype(jnp.bfloat16))
        yg0_ref[...] = yp[:, :DPH]
        yg1_ref[...] = yp[:, DPH:]


def _shared_kernel(x_ref, w1_ref, w3_ref, w2_ref, out_ref, w1b, w3b, w2b):
    e = pl.program_id(0)

    @pl.when(e == 0)
    def _init():
        out_ref[...] = jnp.zeros_like(out_ref)

    w1b[...] = w1_ref[0].astype(jnp.bfloat16)
    w3b[...] = w3_ref[0].astype(jnp.bfloat16)
    w2b[...] = w2_ref[0].astype(jnp.bfloat16)
    xs = x_ref[...].astype(jnp.bfloat16)
    h1 = jnp.dot(xs, w1b[...], preferred_element_type=jnp.float32)
    h3 = jnp.dot(xs, w3b[...], preferred_element_type=jnp.float32)
    h = jax.nn.silu(h1) * h3
    out_ref[...] += jnp.dot(h.astype(jnp.bfloat16), w2b[...],
                            preferred_element_type=jnp.float32)


def _combine_kernel(y0_ref, y1_ref, sh_ref, meta_ref, out_ref):
    w0 = meta_ref[:, 2:3]
    w1 = meta_ref[:, 3:4]
    f32 = jnp.float32
    for q, yref in ((0, y0_ref), (1, y1_ref)):
        lo_a, hi_a = _unpack(yref[0])
        lo_b, hi_b = _unpack(yref[1])
        sl_lo = slice(q * DPH, (q + 1) * DPH)
        sl_hi = slice(DP + q * DPH, DP + (q + 1) * DPH)
        out_ref[:, sl_lo] = (sh_ref[:, sl_lo] + w0 * lo_a.astype(f32) +
                             w1 * lo_b.astype(f32))
        out_ref[:, sl_hi] = (sh_ref[:, sl_hi] + w0 * hi_a.astype(f32) +
                             w1 * hi_b.astype(f32))


def _sc_scatter(xp, p01):
    vmesh = plsc.VectorSubcoreMesh(core_axis_name="c", subcore_axis_name="s")

    @pl.kernel(out_type=tuple(jax.ShapeDtypeStruct((NP, DPH), jnp.int32)
                              for _ in range(2)),
               mesh=vmesh)
    def kern(x_hbm, i_hbm, *xg_hbm):
        for q in range(2):
            dst = xg_hbm[q]

            def body(x_vmem, i_vmem, dst=dst):
                pltpu.sync_copy(x_vmem, dst.at[i_vmem.at[0]])

            pltpu.emit_pipeline(
                body,
                grid=(TOP_K, T // SCW),
                in_specs=[pl.BlockSpec((SCW, DPH), lambda k, i, q=q: (i, q)),
                          pl.BlockSpec((1, SCW), lambda k, i: (k, i))],
                out_specs=[],
                core_axis_name=("c", "s"),
                dimension_semantics=(pltpu.PARALLEL, pltpu.PARALLEL),
            )(x_hbm, i_hbm)

    return kern(xp, p01)


def _sc_gather(ygs, p01):
    vmesh = plsc.VectorSubcoreMesh(core_axis_name="c", subcore_axis_name="s")

    @pl.kernel(out_type=tuple(
        jax.ShapeDtypeStruct((TOP_K, T, DPH), jnp.int32)
        for _ in range(2)),
        mesh=vmesh)
    def kern(yg0, yg1, i_hbm, *y_hbm):
        srcs = (yg0, yg1)
        for q in range(2):
            src = srcs[q]

            def body(i_vmem, o_vmem, src=src):
                pltpu.sync_copy(src.at[i_vmem.at[0]], o_vmem.at[0])

            pltpu.emit_pipeline(
                body,
                grid=(TOP_K, T // SCW),
                in_specs=[pl.BlockSpec((1, SCW), lambda k, i: (k, i))],
                out_specs=[pl.BlockSpec((1, SCW, DPH),
                                        lambda k, i: (k, i, 0))],
                core_axis_name=("c", "s"),
                dimension_semantics=(pltpu.PARALLEL, pltpu.PARALLEL),
            )(i_hbm, y_hbm[q])

    return kern(*ygs, p01)


@functools.partial(jax.jit, static_argnames=("interpret",))
def _moe(x, gate_weight, bias, W1, W3, W2, Ws1, Ws3, Ws2, interpret=False):
    logits = jnp.matmul(x, gate_weight.T)
    lg = jnp.zeros((T, EPAD), jnp.float32).at[:, :E].set(logits)
    bias_row = jnp.zeros((1, EPAD), jnp.float32).at[0, :E].set(bias)

    meta, blk, xp = pl.pallas_call(
        _gate_kernel,
        out_shape=(jax.ShapeDtypeStruct((T, EPAD), jnp.float32),
                   jax.ShapeDtypeStruct((8, EPAD), jnp.float32),
                   jax.ShapeDtypeStruct((T, DP), jnp.int32)),
        interpret=interpret,
    )(lg, bias_row, x)

    p01 = meta[:, :TOP_K].T.astype(jnp.int32)
    emap_s = blk[0, :NB].astype(jnp.int32)
    valid_s = blk[1, :NB].astype(jnp.int32)

    FS = Ws1.shape[1]
    ns = FS // F
    Ws1r = Ws1.reshape(D, ns, F).transpose(1, 0, 2)
    Ws3r = Ws3.reshape(D, ns, F).transpose(1, 0, 2)
    Ws2r = Ws2.reshape(ns, F, D)
    shared = pl.pallas_call(
        _shared_kernel,
        grid=(ns,),
        in_specs=[
            pl.BlockSpec((T, D), lambda e: (0, 0)),
            pl.BlockSpec((1, D, F), lambda e: (e, 0, 0)),
            pl.BlockSpec((1, D, F), lambda e: (e, 0, 0)),
            pl.BlockSpec((1, F, D), lambda e: (e, 0, 0)),
        ],
        out_specs=pl.BlockSpec((T, D), lambda e: (0, 0)),
        out_shape=jax.ShapeDtypeStruct((T, D), jnp.float32),
        scratch_shapes=[
            pltpu.VMEM((D, F), jnp.bfloat16),
            pltpu.VMEM((D, F), jnp.bfloat16),
            pltpu.VMEM((F, D), jnp.bfloat16),
        ],
        interpret=interpret,
    )(x, Ws1r, Ws3r, Ws2r)

    xgs = _sc_scatter(xp, p01)

    xq_spec = pl.BlockSpec((BT, DPH), lambda b, em, va: (b, 0))
    ygs = pl.pallas_call(
        _routed_kernel,
        grid_spec=pltpu.PrefetchScalarGridSpec(
            num_scalar_prefetch=2,
            grid=(NB,),
            in_specs=[
                xq_spec, xq_spec,
                pl.BlockSpec((1, D, F), lambda b, em, va: (em[b], 0, 0)),
                pl.BlockSpec((1, D, F), lambda b, em, va: (em[b], 0, 0)),
                pl.BlockSpec((1, F, D), lambda b, em, va: (em[b], 0, 0)),
            ],
            out_specs=tuple(pl.BlockSpec((BT, DPH), lambda b, em, va: (b, 0))
                            for _ in range(2)),
            scratch_shapes=[
                pltpu.VMEM((D, F), jnp.bfloat16),
                pltpu.VMEM((D, F), jnp.bfloat16),
                pltpu.VMEM((F, D), jnp.bfloat16),
            ],
        ),
        out_shape=tuple(jax.ShapeDtypeStruct((NP, DPH), jnp.int32)
                        for _ in range(2)),
        interpret=interpret,
    )(emap_s, valid_s, *xgs, W1, W3, W2)

    ys = _sc_gather(ygs, p01)

    NTB = 4
    yq_spec = pl.BlockSpec((TOP_K, T // NTB, DPH), lambda t: (0, t, 0))
    out = pl.pallas_call(
        _combine_kernel,
        grid=(NTB,),
        in_specs=[
            yq_spec, yq_spec,
            pl.BlockSpec((T // NTB, D), lambda t: (t, 0)),
            pl.BlockSpec((T // NTB, EPAD), lambda t: (t, 0)),
        ],
        out_specs=pl.BlockSpec((T // NTB, D), lambda t: (t, 0)),
        out_shape=jax.ShapeDtypeStruct((T, D), jnp.float32),
        interpret=interpret,
    )(*ys, shared, meta)
    return out


def kernel(hidden_states, gate_weight, e_score_correction_bias,
           W1, W3, W2, Ws1, Ws3, Ws2):
    return _moe(hidden_states, gate_weight, e_score_correction_bias,
                W1, W3, W2, Ws1, Ws3, Ws2)


# speedup vs baseline: 4.1104x; 1.1518x over previous
import functools

import jax
import jax.numpy as jnp
from jax.experimental import pallas as pl
from jax.experimental.pallas import tpu as pltpu
from jax.experimental.pallas import tpu_sc as plsc

T = 2048
D = 1024
F = 512
E = 8
N_GROUP = 4
TOP_K = 2
TOPK_GROUP = 2
ROUTED_SCALING_FACTOR = 2.5
EPAD = 128
BT = 512
NB = T * TOP_K // BT + E
NP = NB * BT
SCW = 128
DP = D // 2
DPH = DP // 2


def _pack(lo_bf, hi_bf):
    lo = jax.lax.bitcast_convert_type(lo_bf, jnp.uint16).astype(jnp.uint32)
    hi = jax.lax.bitcast_convert_type(hi_bf, jnp.uint16).astype(jnp.uint32)
    return jax.lax.bitcast_convert_type(lo | (hi << 16), jnp.int32)


def _unpack(w):
    u = jax.lax.bitcast_convert_type(w, jnp.uint32)
    lo = jax.lax.bitcast_convert_type((u & 0xFFFF).astype(jnp.uint16),
                                      jnp.bfloat16)
    hi = jax.lax.bitcast_convert_type((u >> 16).astype(jnp.uint16),
                                      jnp.bfloat16)
    return lo, hi


def _gate_kernel(lg_ref, bias_ref, x_ref, meta_ref, blk_ref, xp_ref):
    logits = lg_ref[...]
    scores = jax.nn.sigmoid(logits)
    biased = scores + bias_ref[...]

    lane_r = jax.lax.broadcasted_iota(jnp.int32, (EPAD, EPAD), 0)
    lane_c = jax.lax.broadcasted_iota(jnp.int32, (EPAD, EPAD), 1)
    lane1 = jax.lax.broadcasted_iota(jnp.int32, (1, EPAD), 1)

    f32 = jnp.float32
    hi = jax.lax.Precision.HIGHEST

    def pairwise_topk_mask(vals, n, k):
        A = ((lane_c // n) == lane_r).astype(f32)
        B = ((lane_c % n) == lane_r).astype(f32)
        X = jnp.dot(vals, A, preferred_element_type=f32, precision=hi)
        Y = jnp.dot(vals, B, preferred_element_type=f32, precision=hi)
        tie = ((lane1 % n) < (lane1 // n)).astype(f32)
        valid = (lane1 < n * n) & ((lane1 % n) != (lane1 // n))
        beats = jnp.where((Y > X) | ((Y == X) & (tie > 0)), 1.0, 0.0)
        beats = jnp.where(valid, beats, 0.0)
        Csum = ((lane_r // n) == lane_c).astype(f32) * \
               (lane_r < n * n).astype(f32)
        rank = jnp.dot(beats, Csum, preferred_element_type=f32, precision=hi)
        return jnp.where((rank < k) & (lane1 < n), 1.0, 0.0)

    P = (((lane_r // 2) == lane_c) & (lane_r < E)).astype(f32)
    gscore = jnp.dot(biased, P, preferred_element_type=f32, precision=hi)
    gsel = pairwise_topk_mask(gscore, N_GROUP, TOPK_GROUP)
    Q = ((lane_r == (lane_c // 2)) & (lane_c < E)).astype(f32)
    em = jnp.dot(gsel, Q, preferred_element_type=f32, precision=hi)
    masked = jnp.where((em > 0) & (lane1 < E), biased, -1e9)
    sel = pairwise_topk_mask(masked, E, TOP_K)

    picked = sel * scores
    Ones8 = ((lane_r < E) & (lane_c < E)).astype(f32)
    wsum = jnp.dot(picked, Ones8, preferred_element_type=f32, precision=hi)
    rw = picked * (ROUTED_SCALING_FACTOR / (wsum + 1e-20))

    rT = jax.lax.broadcasted_iota(jnp.int32, (T, T), 0)
    cT = jax.lax.broadcasted_iota(jnp.int32, (T, T), 1)
    Lst = (cT < rT).astype(jnp.bfloat16)
    csum = jnp.dot(Lst, sel.astype(jnp.bfloat16),
                   preferred_element_type=f32)
    counts = jnp.sum(sel, axis=0, keepdims=True)
    pc = jnp.ceil(counts * (1.0 / BT)) * BT
    LTI = ((lane_r <= lane_c) & (lane_r < E)).astype(f32)
    ends = jnp.dot(pc, LTI, preferred_element_type=f32, precision=hi)
    offs = ends - pc
    pos = offs + csum

    lane_f = lane1.astype(f32)
    e0 = jnp.min(jnp.where(sel > 0, lane_f, 1e9), axis=1, keepdims=True)
    e1 = jnp.max(jnp.where(sel > 0, lane_f, -1.0), axis=1, keepdims=True)
    m0 = (lane_f == e0).astype(f32)
    m1 = (lane_f == e1).astype(f32)
    p0 = jnp.sum(pos * m0, axis=1, keepdims=True)
    p1 = jnp.sum(pos * m1, axis=1, keepdims=True)
    w0 = jnp.sum(rw * m0, axis=1, keepdims=True)
    w1 = jnp.sum(rw * m1, axis=1, keepdims=True)

    meta_ref[...] = (p0 * (lane1 == 0) + p1 * (lane1 == 1) +
                     w0 * (lane1 == 2) + w1 * (lane1 == 3))

    bstart = lane_f * BT
    emap = jnp.zeros((1, EPAD), f32)
    for e in range(E):
        emap += (bstart >= ends[:, e:e + 1]).astype(f32)
    total = ends[:, E - 1:E]
    brow = jax.lax.broadcasted_iota(jnp.int32, (8, EPAD), 0)
    blk_ref[...] = jnp.where(brow == 0, jnp.minimum(emap, E - 1.0),
                             jnp.where(brow == 1,
                                       (bstart < total).astype(f32), 0.0))

    xv = x_ref[...]
    xp_ref[...] = _pack(xv[:, :DP].astype(jnp.bfloat16),
                        xv[:, DP:].astype(jnp.bfloat16))


def _routed_kernel(emap_ref, valid_ref, xg0_ref, xg1_ref,
                   w1_ref, w3_ref, w2_ref,
                   yg0_ref, yg1_ref, w1b, w3b, w2b, xfull):
    b = pl.program_id(0)
    changed = (b == 0) | (emap_ref[b] != emap_ref[jnp.maximum(b - 1, 0)])

    @pl.when(changed)
    def _cast():
        w1b[...] = w1_ref[0].astype(jnp.bfloat16)
        w3b[...] = w3_ref[0].astype(jnp.bfloat16)
        w2b[...] = w2_ref[0].astype(jnp.bfloat16)

    @pl.when(valid_ref[b] == 1)
    def _compute():
        f32 = jnp.float32
        lo0, hi0 = _unpack(xg0_ref[...])
        lo1, hi1 = _unpack(xg1_ref[...])
        xfull[:, 0:DPH] = lo0
        xfull[:, DPH:DP] = lo1
        xfull[:, DP:DP + DPH] = hi0
        xfull[:, DP + DPH:] = hi1
        xs = xfull[...]
        h1 = jnp.dot(xs, w1b[...], preferred_element_type=f32)
        h3 = jnp.dot(xs, w3b[...], preferred_element_type=f32)
        h = (jax.nn.silu(h1) * h3).astype(jnp.bfloat16)
        y = jnp.dot(h, w2b[...], preferred_element_type=f32)
        yp = _pack(y[:, :DP].astype(jnp.bfloat16),
                   y[:, DP:].astype(jnp.bfloat16))
        yg0_ref[...] = yp[:, :DPH]
        yg1_ref[...] = yp[:, DPH:]


def _shared_kernel(x_ref, w1_ref, w3_ref, w2_ref, out_ref, w1b, w3b, w2b):
    e = pl.program_id(0)

    @pl.when(e == 0)
    def _init():
        out_ref[...] = jnp.zeros_like(out_ref)

    w1b[...] = w1_ref[...].astype(jnp.bfloat16)
    w3b[...] = w3_ref[...].astype(jnp.bfloat16)
    w2b[...] = w2_ref[...].astype(jnp.bfloat16)
    xs = x_ref[...].astype(jnp.bfloat16)
    h1 = jnp.dot(xs, w1b[...], preferred_element_type=jnp.float32)
    h3 = jnp.dot(xs, w3b[...], preferred_element_type=jnp.float32)
    h = jax.nn.silu(h1) * h3
    out_ref[...] += jnp.dot(h.astype(jnp.bfloat16), w2b[...],
                            preferred_element_type=jnp.float32)


def _combine_kernel(y0_ref, y1_ref, sh_ref, meta_ref, out_ref):
    w0 = meta_ref[:, 2:3]
    w1 = meta_ref[:, 3:4]
    f32 = jnp.float32
    for q, yref in ((0, y0_ref), (1, y1_ref)):
        lo_a, hi_a = _unpack(yref[0])
        lo_b, hi_b = _unpack(yref[1])
        sl_lo = slice(q * DPH, (q + 1) * DPH)
        sl_hi = slice(DP + q * DPH, DP + (q + 1) * DPH)
        out_ref[:, sl_lo] = (sh_ref[:, sl_lo] + w0 * lo_a.astype(f32) +
                             w1 * lo_b.astype(f32))
        out_ref[:, sl_hi] = (sh_ref[:, sl_hi] + w0 * hi_a.astype(f32) +
                             w1 * hi_b.astype(f32))


def _sc_scatter(xp, p01):
    vmesh = plsc.VectorSubcoreMesh(core_axis_name="c", subcore_axis_name="s")

    @pl.kernel(out_type=tuple(jax.ShapeDtypeStruct((NP, DPH), jnp.int32)
                              for _ in range(2)),
               mesh=vmesh)
    def kern(x_hbm, i_hbm, *xg_hbm):
        for q in range(2):
            dst = xg_hbm[q]

            def body(x_vmem, i_vmem, dst=dst):
                pltpu.sync_copy(x_vmem, dst.at[i_vmem.at[0]])

            pltpu.emit_pipeline(
                body,
                grid=(TOP_K, T // SCW),
                in_specs=[pl.BlockSpec((SCW, DPH), lambda k, i, q=q: (i, q)),
                          pl.BlockSpec((1, SCW), lambda k, i: (k, i))],
                out_specs=[],
                core_axis_name=("c", "s"),
                dimension_semantics=(pltpu.PARALLEL, pltpu.PARALLEL),
            )(x_hbm, i_hbm)

    return kern(xp, p01)


def _sc_gather(ygs, p01):
    vmesh = plsc.VectorSubcoreMesh(core_axis_name="c", subcore_axis_name="s")

    @pl.kernel(out_type=tuple(
        jax.ShapeDtypeStruct((TOP_K, T, DPH), jnp.int32)
        for _ in range(2)),
        mesh=vmesh)
    def kern(yg0, yg1, i_hbm, *y_hbm):
        srcs = (yg0, yg1)
        for q in range(2):
            src = srcs[q]

            def body(i_vmem, o_vmem, src=src):
                pltpu.sync_copy(src.at[i_vmem.at[0]], o_vmem.at[0])

            pltpu.emit_pipeline(
                body,
                grid=(TOP_K, T // SCW),
                in_specs=[pl.BlockSpec((1, SCW), lambda k, i: (k, i))],
                out_specs=[pl.BlockSpec((1, SCW, DPH),
                                        lambda k, i: (k, i, 0))],
                core_axis_name=("c", "s"),
                dimension_semantics=(pltpu.PARALLEL, pltpu.PARALLEL),
            )(i_hbm, y_hbm[q])

    return kern(*ygs, p01)


@functools.partial(jax.jit, static_argnames=("interpret",))
def _moe(x, gate_weight, bias, W1, W3, W2, Ws1, Ws3, Ws2, interpret=False):
    logits = jnp.matmul(x, gate_weight.T)
    lg = jnp.zeros((T, EPAD), jnp.float32).at[:, :E].set(logits)
    bias_row = jnp.zeros((1, EPAD), jnp.float32).at[0, :E].set(bias)

    meta, blk, xp = pl.pallas_call(
        _gate_kernel,
        out_shape=(jax.ShapeDtypeStruct((T, EPAD), jnp.float32),
                   jax.ShapeDtypeStruct((8, EPAD), jnp.float32),
                   jax.ShapeDtypeStruct((T, DP), jnp.int32)),
        interpret=interpret,
    )(lg, bias_row, x)

    p01 = meta[:, :TOP_K].T.astype(jnp.int32)
    emap_s = blk[0, :NB].astype(jnp.int32)
    valid_s = blk[1, :NB].astype(jnp.int32)

    FS = Ws1.shape[1]
    ns = FS // F
    shared = pl.pallas_call(
        _shared_kernel,
        grid=(ns,),
        in_specs=[
            pl.BlockSpec((T, D), lambda e: (0, 0)),
            pl.BlockSpec((D, F), lambda e: (0, e)),
            pl.BlockSpec((D, F), lambda e: (0, e)),
            pl.BlockSpec((F, D), lambda e: (e, 0)),
        ],
        out_specs=pl.BlockSpec((T, D), lambda e: (0, 0)),
        out_shape=jax.ShapeDtypeStruct((T, D), jnp.float32),
        scratch_shapes=[
            pltpu.VMEM((D, F), jnp.bfloat16),
            pltpu.VMEM((D, F), jnp.bfloat16),
            pltpu.VMEM((F, D), jnp.bfloat16),
        ],
        interpret=interpret,
    )(x, Ws1, Ws3, Ws2)

    xgs = _sc_scatter(xp, p01)

    xq_spec = pl.BlockSpec((BT, DPH), lambda b, em, va: (b, 0))
    ygs = pl.pallas_call(
        _routed_kernel,
        grid_spec=pltpu.PrefetchScalarGridSpec(
            num_scalar_prefetch=2,
            grid=(NB,),
            in_specs=[
                xq_spec, xq_spec,
                pl.BlockSpec((1, D, F), lambda b, em, va: (em[b], 0, 0)),
                pl.BlockSpec((1, D, F), lambda b, em, va: (em[b], 0, 0)),
                pl.BlockSpec((1, F, D), lambda b, em, va: (em[b], 0, 0)),
            ],
            out_specs=tuple(pl.BlockSpec((BT, DPH), lambda b, em, va: (b, 0))
                            for _ in range(2)),
            scratch_shapes=[
                pltpu.VMEM((D, F), jnp.bfloat16),
                pltpu.VMEM((D, F), jnp.bfloat16),
                pltpu.VMEM((F, D), jnp.bfloat16),
                pltpu.VMEM((BT, D), jnp.bfloat16),
            ],
        ),
        out_shape=tuple(jax.ShapeDtypeStruct((NP, DPH), jnp.int32)
                        for _ in range(2)),
        interpret=interpret,
    )(emap_s, valid_s, *xgs, W1, W3, W2)

    ys = _sc_gather(ygs, p01)

    NTB = 4
    yq_spec = pl.BlockSpec((TOP_K, T // NTB, DPH), lambda t: (0, t, 0))
    out = pl.pallas_call(
        _combine_kernel,
        grid=(NTB,),
        in_specs=[
            yq_spec, yq_spec,
            pl.BlockSpec((T // NTB, D), lambda t: (t, 0)),
            pl.BlockSpec((T // NTB, EPAD), lambda t: (t, 0)),
        ],
        out_specs=pl.BlockSpec((T // NTB, D), lambda t: (t, 0)),
        out_shape=jax.ShapeDtypeStruct((T, D), jnp.float32),
        interpret=interpret,
    )(*ys, shared, meta)
    return out


def kernel(hidden_states, gate_weight, e_score_correction_bias,
           W1, W3, W2, Ws1, Ws3, Ws2):
    return _moe(hidden_states, gate_weight, e_score_correction_bias,
                W1, W3, W2, Ws1, Ws3, Ws2)


# speedup vs baseline: 4.1738x; 1.0154x over previous
import functools

import jax
import jax.numpy as jnp
from jax.experimental import pallas as pl
from jax.experimental.pallas import tpu as pltpu
from jax.experimental.pallas import tpu_sc as plsc

T = 2048
D = 1024
F = 512
E = 8
N_GROUP = 4
TOP_K = 2
TOPK_GROUP = 2
ROUTED_SCALING_FACTOR = 2.5
EPAD = 128
BT = 512
NB = T * TOP_K // BT + E
NP = NB * BT
SCW = 128
DP = D // 2
DPH = DP // 2


def _pack(lo_bf, hi_bf):
    lo = jax.lax.bitcast_convert_type(lo_bf, jnp.uint16).astype(jnp.uint32)
    hi = jax.lax.bitcast_convert_type(hi_bf, jnp.uint16).astype(jnp.uint32)
    return jax.lax.bitcast_convert_type(lo | (hi << 16), jnp.int32)


def _unpack(w):
    u = jax.lax.bitcast_convert_type(w, jnp.uint32)
    lo = jax.lax.bitcast_convert_type((u & 0xFFFF).astype(jnp.uint16),
                                      jnp.bfloat16)
    hi = jax.lax.bitcast_convert_type((u >> 16).astype(jnp.uint16),
                                      jnp.bfloat16)
    return lo, hi


def _gate_kernel(lg_ref, bias_ref, x_ref, meta_ref, blk_ref, xp_ref):
    logits = lg_ref[...]
    scores = jax.nn.sigmoid(logits)
    biased = scores + bias_ref[...]

    lane_r = jax.lax.broadcasted_iota(jnp.int32, (EPAD, EPAD), 0)
    lane_c = jax.lax.broadcasted_iota(jnp.int32, (EPAD, EPAD), 1)
    lane1 = jax.lax.broadcasted_iota(jnp.int32, (1, EPAD), 1)

    f32 = jnp.float32
    hi = jax.lax.Precision.HIGHEST

    def pairwise_topk_mask(vals, n, k):
        A = ((lane_c // n) == lane_r).astype(f32)
        B = ((lane_c % n) == lane_r).astype(f32)
        X = jnp.dot(vals, A, preferred_element_type=f32, precision=hi)
        Y = jnp.dot(vals, B, preferred_element_type=f32, precision=hi)
        tie = ((lane1 % n) < (lane1 // n)).astype(f32)
        valid = (lane1 < n * n) & ((lane1 % n) != (lane1 // n))
        beats = jnp.where((Y > X) | ((Y == X) & (tie > 0)), 1.0, 0.0)
        beats = jnp.where(valid, beats, 0.0)
        Csum = ((lane_r // n) == lane_c).astype(f32) * \
               (lane_r < n * n).astype(f32)
        rank = jnp.dot(beats, Csum, preferred_element_type=f32, precision=hi)
        return jnp.where((rank < k) & (lane1 < n), 1.0, 0.0)

    P = (((lane_r // 2) == lane_c) & (lane_r < E)).astype(f32)
    gscore = jnp.dot(biased, P, preferred_element_type=f32, precision=hi)
    gsel = pairwise_topk_mask(gscore, N_GROUP, TOPK_GROUP)
    Q = ((lane_r == (lane_c // 2)) & (lane_c < E)).astype(f32)
    em = jnp.dot(gsel, Q, preferred_element_type=f32, precision=hi)
    masked = jnp.where((em > 0) & (lane1 < E), biased, -1e9)
    sel = pairwise_topk_mask(masked, E, TOP_K)

    picked = sel * scores
    Ones8 = ((lane_r < E) & (lane_c < E)).astype(f32)
    wsum = jnp.dot(picked, Ones8, preferred_element_type=f32, precision=hi)
    rw = picked * (ROUTED_SCALING_FACTOR / (wsum + 1e-20))

    rT = jax.lax.broadcasted_iota(jnp.int32, (T, T), 0)
    cT = jax.lax.broadcasted_iota(jnp.int32, (T, T), 1)
    Lst = (cT < rT).astype(jnp.bfloat16)
    csum = jnp.dot(Lst, sel.astype(jnp.bfloat16),
                   preferred_element_type=f32)
    counts = jnp.sum(sel, axis=0, keepdims=True)
    pc = jnp.ceil(counts * (1.0 / BT)) * BT
    LTI = ((lane_r <= lane_c) & (lane_r < E)).astype(f32)
    ends = jnp.dot(pc, LTI, preferred_element_type=f32, precision=hi)
    offs = ends - pc
    pos = offs + csum

    lane_f = lane1.astype(f32)
    e0 = jnp.min(jnp.where(sel > 0, lane_f, 1e9), axis=1, keepdims=True)
    e1 = jnp.max(jnp.where(sel > 0, lane_f, -1.0), axis=1, keepdims=True)
    m0 = (lane_f == e0).astype(f32)
    m1 = (lane_f == e1).astype(f32)
    p0 = jnp.sum(pos * m0, axis=1, keepdims=True)
    p1 = jnp.sum(pos * m1, axis=1, keepdims=True)
    w0 = jnp.sum(rw * m0, axis=1, keepdims=True)
    w1 = jnp.sum(rw * m1, axis=1, keepdims=True)

    meta_ref[...] = (p0 * (lane1 == 0) + p1 * (lane1 == 1) +
                     w0 * (lane1 == 2) + w1 * (lane1 == 3))

    bstart = lane_f * BT
    emap = jnp.zeros((1, EPAD), f32)
    for e in range(E):
        emap += (bstart >= ends[:, e:e + 1]).astype(f32)
    total = ends[:, E - 1:E]
    brow = jax.lax.broadcasted_iota(jnp.int32, (8, EPAD), 0)
    blk_ref[...] = jnp.where(brow == 0, jnp.minimum(emap, E - 1.0),
                             jnp.where(brow == 1,
                                       (bstart < total).astype(f32), 0.0))

    xv = x_ref[...]
    xp_ref[...] = _pack(xv[:, :DP].astype(jnp.bfloat16),
                        xv[:, DP:].astype(jnp.bfloat16))


def _routed_kernel(emap_ref, valid_ref, xg0_ref, xg1_ref,
                   w1_ref, w3_ref, w2_ref,
                   yg0_ref, yg1_ref, w1b, w3b, w2b, xfull):
    b = pl.program_id(0)
    changed = (b == 0) | (emap_ref[b] != emap_ref[jnp.maximum(b - 1, 0)])

    @pl.when(changed)
    def _cast():
        w1b[...] = w1_ref[0].astype(jnp.bfloat16)
        w3b[...] = w3_ref[0].astype(jnp.bfloat16)
        w2b[...] = w2_ref[0].astype(jnp.bfloat16)

    @pl.when(valid_ref[b] == 1)
    def _compute():
        f32 = jnp.float32
        lo0, hi0 = _unpack(xg0_ref[...])
        lo1, hi1 = _unpack(xg1_ref[...])
        xfull[:, 0:DPH] = lo0
        xfull[:, DPH:DP] = lo1
        xfull[:, DP:DP + DPH] = hi0
        xfull[:, DP + DPH:] = hi1
        xs = xfull[...]
        h1 = jnp.dot(xs, w1b[...], preferred_element_type=f32)
        h3 = jnp.dot(xs, w3b[...], preferred_element_type=f32)
        h = (jax.nn.silu(h1) * h3).astype(jnp.bfloat16)
        y = jnp.dot(h, w2b[...], preferred_element_type=f32)
        yp = _pack(y[:, :DP].astype(jnp.bfloat16),
                   y[:, DP:].astype(jnp.bfloat16))
        yg0_ref[...] = yp[:, :DPH]
        yg1_ref[...] = yp[:, DPH:]


def _shared_kernel(x_ref, w1_ref, w3_ref, w2_ref, out_ref, w1b, w3b, w2b):
    e = pl.program_id(0)

    @pl.when(e == 0)
    def _init():
        out_ref[...] = jnp.zeros_like(out_ref)

    w1b[...] = w1_ref[...].astype(jnp.bfloat16)
    w3b[...] = w3_ref[...].astype(jnp.bfloat16)
    w2b[...] = w2_ref[...].astype(jnp.bfloat16)
    xs = x_ref[...].astype(jnp.bfloat16)
    h1 = jnp.dot(xs, w1b[...], preferred_element_type=jnp.float32)
    h3 = jnp.dot(xs, w3b[...], preferred_element_type=jnp.float32)
    h = jax.nn.silu(h1) * h3
    out_ref[...] += jnp.dot(h.astype(jnp.bfloat16), w2b[...],
                            preferred_element_type=jnp.float32)


def _combine_kernel(y0_ref, y1_ref, sh_ref, meta_ref, out_ref):
    w0 = meta_ref[:, 2:3]
    w1 = meta_ref[:, 3:4]
    f32 = jnp.float32
    for q, yref in ((0, y0_ref), (1, y1_ref)):
        lo_a, hi_a = _unpack(yref[0])
        lo_b, hi_b = _unpack(yref[1])
        sl_lo = slice(q * DPH, (q + 1) * DPH)
        sl_hi = slice(DP + q * DPH, DP + (q + 1) * DPH)
        out_ref[:, sl_lo] = (sh_ref[:, sl_lo] + w0 * lo_a.astype(f32) +
                             w1 * lo_b.astype(f32))
        out_ref[:, sl_hi] = (sh_ref[:, sl_hi] + w0 * hi_a.astype(f32) +
                             w1 * hi_b.astype(f32))


def _sc_scatter(xp, p01):
    vmesh = plsc.VectorSubcoreMesh(core_axis_name="c", subcore_axis_name="s")

    @pl.kernel(out_type=tuple(jax.ShapeDtypeStruct((NP, DPH), jnp.int32)
                              for _ in range(2)),
               mesh=vmesh)
    def kern(x_hbm, i_hbm, *xg_hbm):
        for q in range(2):
            dst = xg_hbm[q]

            def body(x_vmem, i_vmem, dst=dst):
                pltpu.sync_copy(x_vmem, dst.at[i_vmem.at[0]])

            pltpu.emit_pipeline(
                body,
                grid=(TOP_K, T // SCW),
                in_specs=[pl.BlockSpec((SCW, DPH), lambda k, i, q=q: (i, q)),
                          pl.BlockSpec((1, SCW), lambda k, i: (k, i))],
                out_specs=[],
                core_axis_name=("c", "s"),
                dimension_semantics=(pltpu.PARALLEL, pltpu.PARALLEL),
            )(x_hbm, i_hbm)

    return kern(xp, p01)


def _sc_gather(ygs, p01):
    vmesh = plsc.VectorSubcoreMesh(core_axis_name="c", subcore_axis_name="s")

    @pl.kernel(out_type=tuple(
        jax.ShapeDtypeStruct((TOP_K, T, DPH), jnp.int32)
        for _ in range(2)),
        mesh=vmesh)
    def kern(yg0, yg1, i_hbm, *y_hbm):
        srcs = (yg0, yg1)
        for q in range(2):
            src = srcs[q]

            def body(i_vmem, o_vmem, src=src):
                pltpu.sync_copy(src.at[i_vmem.at[0]], o_vmem.at[0])

            pltpu.emit_pipeline(
                body,
                grid=(TOP_K, T // SCW),
                in_specs=[pl.BlockSpec((1, SCW), lambda k, i: (k, i))],
                out_specs=[pl.BlockSpec((1, SCW, DPH),
                                        lambda k, i: (k, i, 0))],
                core_axis_name=("c", "s"),
                dimension_semantics=(pltpu.PARALLEL, pltpu.PARALLEL),
            )(i_hbm, y_hbm[q])

    return kern(*ygs, p01)


@jax.jit
def _moe(x, gate_weight, bias, W1, W3, W2, Ws1, Ws3, Ws2):
    logits = jnp.matmul(x, gate_weight.T)
    lg = jnp.zeros((T, EPAD), jnp.float32).at[:, :E].set(logits)
    bias_row = jnp.zeros((1, EPAD), jnp.float32).at[0, :E].set(bias)

    meta, blk, xp = pl.pallas_call(
        _gate_kernel,
        out_shape=(jax.ShapeDtypeStruct((T, EPAD), jnp.float32),
                   jax.ShapeDtypeStruct((8, EPAD), jnp.float32),
                   jax.ShapeDtypeStruct((T, DP), jnp.int32)),
    )(lg, bias_row, x)

    p01 = meta[:, :TOP_K].T.astype(jnp.int32)
    emap_s = blk[0, :NB].astype(jnp.int32)
    valid_s = blk[1, :NB].astype(jnp.int32)

    FS = Ws1.shape[1]
    ns = FS // F
    shared = pl.pallas_call(
        _shared_kernel,
        grid=(ns,),
        in_specs=[
            pl.BlockSpec((T, D), lambda e: (0, 0)),
            pl.BlockSpec((D, F), lambda e: (0, e)),
            pl.BlockSpec((D, F), lambda e: (0, e)),
            pl.BlockSpec((F, D), lambda e: (e, 0)),
        ],
        out_specs=pl.BlockSpec((T, D), lambda e: (0, 0)),
        out_shape=jax.ShapeDtypeStruct((T, D), jnp.float32),
        scratch_shapes=[
            pltpu.VMEM((D, F), jnp.bfloat16),
            pltpu.VMEM((D, F), jnp.bfloat16),
            pltpu.VMEM((F, D), jnp.bfloat16),
        ],
    )(x, Ws1, Ws3, Ws2)

    xgs = _sc_scatter(xp, p01)

    xq_spec = pl.BlockSpec((BT, DPH), lambda b, em, va: (b, 0))
    ygs = pl.pallas_call(
        _routed_kernel,
        grid_spec=pltpu.PrefetchScalarGridSpec(
            num_scalar_prefetch=2,
            grid=(NB,),
            in_specs=[
                xq_spec, xq_spec,
                pl.BlockSpec((1, D, F), lambda b, em, va: (em[b], 0, 0)),
                pl.BlockSpec((1, D, F), lambda b, em, va: (em[b], 0, 0)),
                pl.BlockSpec((1, F, D), lambda b, em, va: (em[b], 0, 0)),
            ],
            out_specs=tuple(pl.BlockSpec((BT, DPH), lambda b, em, va: (b, 0))
                            for _ in range(2)),
            scratch_shapes=[
                pltpu.VMEM((D, F), jnp.bfloat16),
                pltpu.VMEM((D, F), jnp.bfloat16),
                pltpu.VMEM((F, D), jnp.bfloat16),
                pltpu.VMEM((BT, D), jnp.bfloat16),
            ],
        ),
        out_shape=tuple(jax.ShapeDtypeStruct((NP, DPH), jnp.int32)
                        for _ in range(2)),
    )(emap_s, valid_s, *xgs, W1, W3, W2)

    ys = _sc_gather(ygs, p01)

    NTB = 4
    yq_spec = pl.BlockSpec((TOP_K, T // NTB, DPH), lambda t: (0, t, 0))
    out = pl.pallas_call(
        _combine_kernel,
        grid=(NTB,),
        in_specs=[
            yq_spec, yq_spec,
            pl.BlockSpec((T // NTB, D), lambda t: (t, 0)),
            pl.BlockSpec((T // NTB, EPAD), lambda t: (t, 0)),
        ],
        out_specs=pl.BlockSpec((T // NTB, D), lambda t: (t, 0)),
        out_shape=jax.ShapeDtypeStruct((T, D), jnp.float32),
    )(*ys, shared, meta)
    return out


def kernel(hidden_states, gate_weight, e_score_correction_bias,
           W1, W3, W2, Ws1, Ws3, Ws2):
    return _moe(hidden_states, gate_weight, e_score_correction_bias,
                W1, W3, W2, Ws1, Ws3, Ws2)
